# baseline (speedup 1.0000x reference)
import functools

import jax
import jax.numpy as jnp
from jax import lax
from jax.experimental import pallas as pl
from jax.experimental.pallas import tpu as pltpu
from jax.experimental.pallas import tpu_sc as plsc

B, N, F = 4, 2500, 128
E = 80000
D = 128
NUM_ACTIONS = 8
STEPS = 3

BN = B * N
BE = B * E

NC, NS = 2, 16
NW = NC * NS
CHUNK = 128
NCH = 80
EPT = NCH * CHUNK
BEP = NW * EPT
KBUF = 1
GS = 256
IDXC = 16
BNP = 10240
PAD_DST = BN
ROWS_PER_TILE = BNP // NS
ZROWS = 128


def _mish(x):
    sp = jnp.maximum(x, 0.0) + jnp.log1p(jnp.exp(-jnp.abs(x)))
    return x * jnp.tanh(sp)


def _feat_body(x_ref, w_ref, b_ref, o_ref):
    t = jnp.dot(x_ref[...], w_ref[...], preferred_element_type=jnp.float32)
    o_ref[...] = _mish(t + b_ref[...])


def _update_body(h_ref, p_ref, w1_ref, w2_ref, b_ref, o_ref):
    agg = p_ref[0] + p_ref[1]
    t = (jnp.dot(h_ref[...], w1_ref[...], preferred_element_type=jnp.float32)
         + jnp.dot(agg, w2_ref[...], preferred_element_type=jnp.float32)
         + b_ref[...])
    o_ref[...] = _mish(t)


def _heads_body(h_ref, m1_ref, m0_ref, wa1_ref, ba1_ref, wa2_ref, ba2_ref, o_ref):
    hg = h_ref[0]
    x1 = jnp.dot(hg, wa1_ref[...], preferred_element_type=jnp.float32)
    x1 = x1 + ba1_ref[0, 0]
    m1 = m1_ref[0]
    x1m = jnp.where(m1 > 0.0, x1, -1e9)
    maxv = jnp.max(x1m)
    iota = lax.broadcasted_iota(jnp.int32, (N, 1), 0)
    idx = jnp.min(jnp.where(x1m == maxv, iota, jnp.int32(2**30)))
    x2 = jnp.dot(hg, wa2_ref[...], preferred_element_type=jnp.float32)
    x2 = x2 + ba2_ref[...]
    iota2 = lax.broadcasted_iota(jnp.int32, (N, NUM_ACTIONS), 0)
    sel = jnp.sum(jnp.where(iota2 == idx, x2, 0.0), axis=0, keepdims=True)
    selm = jnp.where(m0_ref[0] > 0.0, sel, -1e9)
    o_ref[0] = jnp.concatenate([maxv.reshape(1, 1), selm], axis=1)


def _tc_feat(x, w, b2):
    return pl.pallas_call(
        _feat_body,
        grid=(5,),
        in_specs=[
            pl.BlockSpec((2000, F), lambda i: (i, 0)),
            pl.BlockSpec((F, D), lambda i: (0, 0)),
            pl.BlockSpec((1, D), lambda i: (0, 0)),
        ],
        out_specs=pl.BlockSpec((2000, D), lambda i: (i, 0)),
        out_shape=jax.ShapeDtypeStruct((BN, D), jnp.float32),
    )(x, w, b2)


def _tc_update(h, parts, w1, w2, b2):
    return pl.pallas_call(
        _update_body,
        grid=(5,),
        in_specs=[
            pl.BlockSpec((2000, D), lambda i: (i, 0)),
            pl.BlockSpec((2, 2000, D), lambda i: (0, i, 0)),
            pl.BlockSpec((D, D), lambda i: (0, 0)),
            pl.BlockSpec((D, D), lambda i: (0, 0)),
            pl.BlockSpec((1, D), lambda i: (0, 0)),
        ],
        out_specs=pl.BlockSpec((2000, D), lambda i: (i, 0)),
        out_shape=jax.ShapeDtypeStruct((BN, D), jnp.float32),
    )(h, parts, w1, w2, b2)


def _tc_heads(h3, m1, m0, wa1, ba1, wa2, ba2):
    return pl.pallas_call(
        _heads_body,
        grid=(B,),
        in_specs=[
            pl.BlockSpec((1, N, D), lambda i: (i, 0, 0)),
            pl.BlockSpec((1, N, 1), lambda i: (i, 0, 0)),
            pl.BlockSpec((1, 1, NUM_ACTIONS), lambda i: (i, 0, 0)),
            pl.BlockSpec((D, 1), lambda i: (0, 0)),
            pl.BlockSpec((1, 1), lambda i: (0, 0)),
            pl.BlockSpec((D, NUM_ACTIONS), lambda i: (0, 0)),
            pl.BlockSpec((1, NUM_ACTIONS), lambda i: (0, 0)),
        ],
        out_specs=pl.BlockSpec((1, 1, 1 + NUM_ACTIONS), lambda i: (i, 0, 0)),
        out_shape=jax.ShapeDtypeStruct((B, 1, 1 + NUM_ACTIONS), jnp.float32),
    )(h3, m1, m0, wa1, ba1, wa2, ba2)


def _segsum_body(h_hbm, srcf_hbm, dst_hbm, out_hbm,
                 agg_sh, srcbuf, dstbuf, rows, sems):
    cid = lax.axis_index("c")
    sid = lax.axis_index("s")
    wid = sid * NC + cid

    zero = jnp.zeros((16,), jnp.float32)

    zstage = rows[0].at[pl.ds(0, ZROWS)]

    def zloop(i, _):
        for j in range(D // 16):
            rows[0][i, pl.ds(j * 16, 16)] = zero
        return ()

    lax.fori_loop(0, ZROWS, zloop, ())
    for k in range(ROWS_PER_TILE // ZROWS):
        pltpu.sync_copy(zstage,
                        agg_sh.at[pl.ds(sid * ROWS_PER_TILE + k * ZROWS, ZROWS)])
    plsc.subcore_barrier()

    def window(w, _):
        wa = pl.multiple_of(w * IDXC, IDXC)
        wf = pl.multiple_of(w * IDXC * CHUNK, IDXC * CHUNK)
        pltpu.sync_copy(srcf_hbm.at[wid, pl.ds(wf, IDXC * CHUNK)], srcbuf)
        pltpu.sync_copy(dst_hbm.at[wid, pl.ds(wa, IDXC)], dstbuf)
        for g in range(IDXC * CHUNK // GS):
            d = pltpu.async_copy(
                h_hbm.at[srcbuf.at[pl.ds(g * GS, GS)]], rows[0], sems[0])
            d.wait()
            for t in range(GS // CHUNK):
                pltpu.sync_copy(
                    rows[0].at[pl.ds(t * CHUNK, CHUNK)],
                    agg_sh.at[dstbuf.at[g * (GS // CHUNK) + t]], add=True)
        return ()

    lax.fori_loop(0, NCH // IDXC, window, ())
    plsc.subcore_barrier()

    for k in range(ROWS_PER_TILE // ZROWS):
        r0 = sid * ROWS_PER_TILE + k * ZROWS
        pltpu.sync_copy(agg_sh.at[pl.ds(r0, ZROWS)], zstage)
        pltpu.sync_copy(zstage, out_hbm.at[cid].at[pl.ds(r0, ZROWS)])


def _sc_segsum(h, srcf, dst):
    mesh = plsc.VectorSubcoreMesh(core_axis_name="c", subcore_axis_name="s",
                                  num_cores=NC, num_subcores=NS)
    fn = pl.kernel(
        _segsum_body,
        out_type=jax.ShapeDtypeStruct((NC, BNP, D), jnp.float32),
        mesh=mesh,
        scratch_types=[
            pltpu.VMEM_SHARED((BNP, D), jnp.float32),
            pltpu.VMEM((IDXC * CHUNK,), jnp.int32),
            pltpu.VMEM((IDXC, CHUNK), jnp.int32),
            [pltpu.VMEM((GS, D), jnp.float32) for _ in range(KBUF)],
            [pltpu.SemaphoreType.DMA for _ in range(KBUF)],
        ],
    )
    return fn(h, srcf, dst)


def kernel(nodes, edge_index, mask_0, mask_1, W_feat, b_feat, W_upd, b_upd,
           W_glob, b_glob, Wa1, ba1, Wa2, ba2):
    x = nodes.reshape(BN, F)
    offs = (jnp.arange(B, dtype=edge_index.dtype) * N).reshape(B, 1, 1)
    ei = (edge_index + offs).astype(jnp.int32)
    npad = BEP - BE
    srcf = jnp.concatenate(
        [ei[..., 0].reshape(BE), jnp.zeros((npad,), jnp.int32)]
    ).reshape(NW, EPT)
    dst = jnp.concatenate(
        [ei[..., 1].reshape(BE), jnp.full((npad,), PAD_DST, jnp.int32)]
    ).reshape(NW, NCH, CHUNK)

    h = _tc_feat(x, W_feat, b_feat.reshape(1, D))
    for s in range(STEPS):
        parts = _sc_segsum(h, srcf, dst)
        h = _tc_update(h, parts, W_upd[s, :D, :], W_upd[s, D:, :],
                       b_upd[s].reshape(1, D))

    h3 = h.reshape(B, N, D)
    m1 = mask_1.astype(jnp.float32).reshape(B, N, 1)
    m0 = mask_0.astype(jnp.float32).reshape(B, 1, NUM_ACTIONS)
    out = _tc_heads(h3, m1, m0, Wa1, ba1.reshape(1, 1), Wa2,
                    ba2.reshape(1, NUM_ACTIONS))
    return out.reshape(B, 1 + NUM_ACTIONS)


# speedup vs baseline: 3.0530x; 3.0530x over previous
import jax
import jax.numpy as jnp
from jax import lax
from jax.experimental import pallas as pl
from jax.experimental.pallas import tpu as pltpu
from jax.experimental.pallas import tpu_sc as plsc

B, N, F = 4, 2500, 128
E = 80000
D = 128
NUM_ACTIONS = 8
STEPS = 3

NP = 2560
BNP = B * NP
BE = B * E

NC, NS = 2, 16
NW = NC * NS
GPC = B // NC
HR = GPC * NP
SROWS = HR // NS
CHUNK = 128
NCH = 80
EPT = NCH * CHUNK
EPC = NS * EPT
KBUF = 2
IDXC = 16
PAD_DST = N


def _mish(x):
    sp = jnp.maximum(x, 0.0) + jnp.log1p(jnp.exp(-jnp.abs(x)))
    return x * jnp.tanh(sp)


def _feat_body(x_ref, w_ref, b_ref, o_ref):
    t = jnp.dot(x_ref[...], w_ref[...], preferred_element_type=jnp.float32)
    o_ref[...] = _mish(t + b_ref[...])


def _update_body(h_ref, a_ref, w1_ref, w2_ref, b_ref, o_ref):
    t = (jnp.dot(h_ref[...], w1_ref[...], preferred_element_type=jnp.float32)
         + jnp.dot(a_ref[...], w2_ref[...], preferred_element_type=jnp.float32)
         + b_ref[...])
    o_ref[...] = _mish(t)


def _heads_body(h_ref, m1_ref, m0_ref, wa1_ref, ba1_ref, wa2_ref, ba2_ref, o_ref):
    hg = h_ref[0]
    x1 = jnp.dot(hg, wa1_ref[...], preferred_element_type=jnp.float32)
    x1 = x1 + ba1_ref[0, 0]
    m1 = m1_ref[0]
    x1m = jnp.where(m1 > 0.0, x1, -1e9)
    maxv = jnp.max(x1m)
    iota = lax.broadcasted_iota(jnp.int32, (NP, 1), 0)
    idx = jnp.min(jnp.where(x1m == maxv, iota, jnp.int32(2**30)))
    x2 = jnp.dot(hg, wa2_ref[...], preferred_element_type=jnp.float32)
    x2 = x2 + ba2_ref[...]
    iota2 = lax.broadcasted_iota(jnp.int32, (NP, NUM_ACTIONS), 0)
    sel = jnp.sum(jnp.where(iota2 == idx, x2, 0.0), axis=0, keepdims=True)
    selm = jnp.where(m0_ref[0] > 0.0, sel, -1e9)
    o_ref[0] = jnp.concatenate([maxv.reshape(1, 1), selm], axis=1)


def _tc_feat(x, w, b2):
    return pl.pallas_call(
        _feat_body,
        grid=(5,),
        in_specs=[
            pl.BlockSpec((2048, F), lambda i: (i, 0)),
            pl.BlockSpec((F, D), lambda i: (0, 0)),
            pl.BlockSpec((1, D), lambda i: (0, 0)),
        ],
        out_specs=pl.BlockSpec((2048, D), lambda i: (i, 0)),
        out_shape=jax.ShapeDtypeStruct((BNP, D), jnp.float32),
    )(x, w, b2)


def _tc_update(h, agg, w1, w2, b2):
    return pl.pallas_call(
        _update_body,
        grid=(5,),
        in_specs=[
            pl.BlockSpec((2048, D), lambda i: (i, 0)),
            pl.BlockSpec((2048, D), lambda i: (i, 0)),
            pl.BlockSpec((D, D), lambda i: (0, 0)),
            pl.BlockSpec((D, D), lambda i: (0, 0)),
            pl.BlockSpec((1, D), lambda i: (0, 0)),
        ],
        out_specs=pl.BlockSpec((2048, D), lambda i: (i, 0)),
        out_shape=jax.ShapeDtypeStruct((BNP, D), jnp.float32),
    )(h, agg, w1, w2, b2)


def _tc_heads(h3, m1, m0, wa1, ba1, wa2, ba2):
    return pl.pallas_call(
        _heads_body,
        grid=(B,),
        in_specs=[
            pl.BlockSpec((1, NP, D), lambda i: (i, 0, 0)),
            pl.BlockSpec((1, NP, 1), lambda i: (i, 0, 0)),
            pl.BlockSpec((1, 1, NUM_ACTIONS), lambda i: (i, 0, 0)),
            pl.BlockSpec((D, 1), lambda i: (0, 0)),
            pl.BlockSpec((1, 1), lambda i: (0, 0)),
            pl.BlockSpec((D, NUM_ACTIONS), lambda i: (0, 0)),
            pl.BlockSpec((1, NUM_ACTIONS), lambda i: (0, 0)),
        ],
        out_specs=pl.BlockSpec((1, 1, 1 + NUM_ACTIONS), lambda i: (i, 0, 0)),
        out_shape=jax.ShapeDtypeStruct((B, 1, 1 + NUM_ACTIONS), jnp.float32),
    )(h3, m1, m0, wa1, ba1, wa2, ba2)


def _segsum_body(h_hbm, src_hbm, dst_hbm, out_hbm,
                 hloc_sh, agg_sh, srcbuf, dstbuf, rows, sems):
    cid = lax.axis_index("c")
    sid = lax.axis_index("s")
    wid = sid * NC + cid

    pltpu.sync_copy(h_hbm.at[pl.ds(cid * HR + sid * SROWS, SROWS)],
                    hloc_sh.at[pl.ds(sid * SROWS, SROWS)])

    zero = jnp.zeros((16,), jnp.float32)

    def zloop(i, _):
        for j in range(D // 16):
            rows[0][i, pl.ds(j * 16, 16)] = zero
        return ()

    lax.fori_loop(0, CHUNK, zloop, ())
    for k in range(SROWS // CHUNK):
        pltpu.sync_copy(rows[0],
                        agg_sh.at[pl.ds(sid * SROWS + k * CHUNK, CHUNK)])
    even = SROWS - (SROWS // CHUNK) * CHUNK
    if even:
        pltpu.sync_copy(
            rows[0].at[pl.ds(0, even)],
            agg_sh.at[pl.ds(sid * SROWS + (SROWS // CHUNK) * CHUNK, even)])
    plsc.subcore_barrier()

    def window(w, _):
        wa = pl.multiple_of(w * IDXC, IDXC)
        pltpu.sync_copy(src_hbm.at[wid, pl.ds(wa, IDXC)], srcbuf)
        pltpu.sync_copy(dst_hbm.at[wid, pl.ds(wa, IDXC)], dstbuf)
        descs = [
            pltpu.async_copy(hloc_sh.at[srcbuf.at[j]], rows[j], sems[j])
            for j in range(KBUF)
        ]
        for j in range(IDXC):
            descs[j].wait()
            pltpu.sync_copy(rows[j % KBUF], agg_sh.at[dstbuf.at[j]], add=True)
            if j + KBUF < IDXC:
                descs.append(
                    pltpu.async_copy(hloc_sh.at[srcbuf.at[j + KBUF]],
                                     rows[(j + KBUF) % KBUF],
                                     sems[(j + KBUF) % KBUF]))
        return ()

    lax.fori_loop(0, NCH // IDXC, window, ())
    plsc.subcore_barrier()

    pltpu.sync_copy(agg_sh.at[pl.ds(sid * SROWS, SROWS)],
                    out_hbm.at[cid].at[pl.ds(sid * SROWS, SROWS)])


def _sc_segsum(h, src, dst):
    mesh = plsc.VectorSubcoreMesh(core_axis_name="c", subcore_axis_name="s",
                                  num_cores=NC, num_subcores=NS)
    fn = pl.kernel(
        _segsum_body,
        out_type=jax.ShapeDtypeStruct((NC, HR, D), jnp.float32),
        mesh=mesh,
        scratch_types=[
            pltpu.VMEM_SHARED((HR, D), jnp.float32),
            pltpu.VMEM_SHARED((HR, D), jnp.float32),
            pltpu.VMEM((IDXC, CHUNK), jnp.int32),
            pltpu.VMEM((IDXC, CHUNK), jnp.int32),
            [pltpu.VMEM((CHUNK, D), jnp.float32) for _ in range(KBUF)],
            [pltpu.SemaphoreType.DMA for _ in range(KBUF)],
        ],
    )
    return fn(h, src, dst)


def _edge_arrays(edge_index):
    ei = edge_index.astype(jnp.int32)
    goff = (jnp.arange(B, dtype=jnp.int32) % GPC * NP).reshape(B, 1)
    srcl = ei[..., 0] + goff
    dstl = ei[..., 1] + goff
    npad = EPC - GPC * E

    def split(a, padval):
        a2 = a.reshape(NC, GPC * E)
        a2 = jnp.concatenate(
            [a2, jnp.full((NC, npad), padval, jnp.int32)], axis=1)
        return (a2.reshape(NC, NS, EPT).transpose(1, 0, 2)
                  .reshape(NW, NCH, CHUNK))

    return split(srcl, 0), split(dstl, PAD_DST)


def kernel(nodes, edge_index, mask_0, mask_1, W_feat, b_feat, W_upd, b_upd,
           W_glob, b_glob, Wa1, ba1, Wa2, ba2):
    x = jnp.pad(nodes, ((0, 0), (0, NP - N), (0, 0))).reshape(BNP, F)
    src, dst = _edge_arrays(edge_index)

    h = _tc_feat(x, W_feat, b_feat.reshape(1, D))
    for s in range(STEPS):
        parts = _sc_segsum(h, src, dst)
        h = _tc_update(h, parts.reshape(BNP, D), W_upd[s, :D, :],
                       W_upd[s, D:, :], b_upd[s].reshape(1, D))

    h3 = h.reshape(B, NP, D)
    m1 = jnp.pad(mask_1.astype(jnp.float32), ((0, 0), (0, NP - N))
                 ).reshape(B, NP, 1)
    m0 = mask_0.astype(jnp.float32).reshape(B, 1, NUM_ACTIONS)
    out = _tc_heads(h3, m1, m0, Wa1, ba1.reshape(1, 1), Wa2,
                    ba2.reshape(1, NUM_ACTIONS))
    return out.reshape(B, 1 + NUM_ACTIONS)


# speedup vs baseline: 3.2392x; 1.0610x over previous
import jax
import jax.numpy as jnp
from jax import lax
from jax.experimental import pallas as pl
from jax.experimental.pallas import tpu as pltpu
from jax.experimental.pallas import tpu_sc as plsc

B, N, F = 4, 2500, 128
E = 80000
D = 128
NUM_ACTIONS = 8
STEPS = 3

NP = 2560
BNP = B * NP
BE = B * E

NC, NS = 2, 16
NW = NC * NS
GPC = B // NC
HR = GPC * NP
SROWS = HR // NS
CHUNK = 128
NCH = 80
EPT = NCH * CHUNK
EPC = NS * EPT
KBUF = 2
IDXC = 40
PAD_DST = N


def _mish(x):
    sp = jnp.maximum(x, 0.0) + jnp.log1p(jnp.exp(-jnp.abs(x)))
    return x * jnp.tanh(sp)


def _feat_body(x_ref, w_ref, b_ref, o_ref):
    t = jnp.dot(x_ref[...], w_ref[...], preferred_element_type=jnp.float32)
    o_ref[...] = _mish(t + b_ref[...])


def _update_body(h_ref, a_ref, w1_ref, w2_ref, b_ref, o_ref):
    t = (jnp.dot(h_ref[...], w1_ref[...], preferred_element_type=jnp.float32)
         + jnp.dot(a_ref[...], w2_ref[...], preferred_element_type=jnp.float32)
         + b_ref[...])
    o_ref[...] = _mish(t)


def _heads_body(h_ref, m1_ref, m0_ref, wa1_ref, ba1_ref, wa2_ref, ba2_ref, o_ref):
    hg = h_ref[0]
    x1 = jnp.dot(hg, wa1_ref[...], preferred_element_type=jnp.float32)
    x1 = x1 + ba1_ref[0, 0]
    m1 = m1_ref[0]
    x1m = jnp.where(m1 > 0.0, x1, -1e9)
    maxv = jnp.max(x1m)
    iota = lax.broadcasted_iota(jnp.int32, (NP, 1), 0)
    idx = jnp.min(jnp.where(x1m == maxv, iota, jnp.int32(2**30)))
    x2 = jnp.dot(hg, wa2_ref[...], preferred_element_type=jnp.float32)
    x2 = x2 + ba2_ref[...]
    iota2 = lax.broadcasted_iota(jnp.int32, (NP, NUM_ACTIONS), 0)
    sel = jnp.sum(jnp.where(iota2 == idx, x2, 0.0), axis=0, keepdims=True)
    selm = jnp.where(m0_ref[0] > 0.0, sel, -1e9)
    o_ref[0] = jnp.concatenate([maxv.reshape(1, 1), selm], axis=1)


def _tc_feat(x, w, b2):
    return pl.pallas_call(
        _feat_body,
        grid=(5,),
        in_specs=[
            pl.BlockSpec((2048, F), lambda i: (i, 0)),
            pl.BlockSpec((F, D), lambda i: (0, 0)),
            pl.BlockSpec((1, D), lambda i: (0, 0)),
        ],
        out_specs=pl.BlockSpec((2048, D), lambda i: (i, 0)),
        out_shape=jax.ShapeDtypeStruct((BNP, D), jnp.float32),
    )(x, w, b2)


def _tc_update(h, agg, w1, w2, b2):
    return pl.pallas_call(
        _update_body,
        grid=(5,),
        in_specs=[
            pl.BlockSpec((2048, D), lambda i: (i, 0)),
            pl.BlockSpec((2048, D), lambda i: (i, 0)),
            pl.BlockSpec((D, D), lambda i: (0, 0)),
            pl.BlockSpec((D, D), lambda i: (0, 0)),
            pl.BlockSpec((1, D), lambda i: (0, 0)),
        ],
        out_specs=pl.BlockSpec((2048, D), lambda i: (i, 0)),
        out_shape=jax.ShapeDtypeStruct((BNP, D), jnp.float32),
    )(h, agg, w1, w2, b2)


def _tc_heads(h3, m1, m0, wa1, ba1, wa2, ba2):
    return pl.pallas_call(
        _heads_body,
        grid=(B,),
        in_specs=[
            pl.BlockSpec((1, NP, D), lambda i: (i, 0, 0)),
            pl.BlockSpec((1, NP, 1), lambda i: (i, 0, 0)),
            pl.BlockSpec((1, 1, NUM_ACTIONS), lambda i: (i, 0, 0)),
            pl.BlockSpec((D, 1), lambda i: (0, 0)),
            pl.BlockSpec((1, 1), lambda i: (0, 0)),
            pl.BlockSpec((D, NUM_ACTIONS), lambda i: (0, 0)),
            pl.BlockSpec((1, NUM_ACTIONS), lambda i: (0, 0)),
        ],
        out_specs=pl.BlockSpec((1, 1, 1 + NUM_ACTIONS), lambda i: (i, 0, 0)),
        out_shape=jax.ShapeDtypeStruct((B, 1, 1 + NUM_ACTIONS), jnp.float32),
    )(h3, m1, m0, wa1, ba1, wa2, ba2)


def _segsum_body(h_hbm, src_hbm, dst_hbm, out_hbm,
                 hloc_sh, agg_sh, srcbuf, dstbuf, rows, sems):
    cid = lax.axis_index("c")
    sid = lax.axis_index("s")
    wid = sid * NC + cid

    hstage = pltpu.async_copy(h_hbm.at[pl.ds(cid * HR + sid * SROWS, SROWS)],
                              hloc_sh.at[pl.ds(sid * SROWS, SROWS)], sems[0])

    zero = jnp.zeros((16,), jnp.float32)

    def zloop(i, _):
        for j in range(D // 16):
            rows[0][i, pl.ds(j * 16, 16)] = zero
        return ()

    lax.fori_loop(0, CHUNK, zloop, ())
    hstage.wait()
    for k in range(SROWS // CHUNK):
        pltpu.sync_copy(rows[0],
                        agg_sh.at[pl.ds(sid * SROWS + k * CHUNK, CHUNK)])
    even = SROWS - (SROWS // CHUNK) * CHUNK
    if even:
        pltpu.sync_copy(
            rows[0].at[pl.ds(0, even)],
            agg_sh.at[pl.ds(sid * SROWS + (SROWS // CHUNK) * CHUNK, even)])
    plsc.subcore_barrier()

    def window(w, _):
        wa = pl.multiple_of(w * IDXC, IDXC)
        pltpu.sync_copy(src_hbm.at[wid, pl.ds(wa, IDXC)], srcbuf)
        pltpu.sync_copy(dst_hbm.at[wid, pl.ds(wa, IDXC)], dstbuf)
        descs = [
            pltpu.async_copy(hloc_sh.at[srcbuf.at[j]], rows[j], sems[j])
            for j in range(KBUF)
        ]
        for j in range(IDXC):
            descs[j].wait()
            pltpu.sync_copy(rows[j % KBUF], agg_sh.at[dstbuf.at[j]], add=True)
            if j + KBUF < IDXC:
                descs.append(
                    pltpu.async_copy(hloc_sh.at[srcbuf.at[j + KBUF]],
                                     rows[(j + KBUF) % KBUF],
                                     sems[(j + KBUF) % KBUF]))
        return ()

    lax.fori_loop(0, NCH // IDXC, window, ())
    plsc.subcore_barrier()

    pltpu.sync_copy(agg_sh.at[pl.ds(sid * SROWS, SROWS)],
                    out_hbm.at[cid].at[pl.ds(sid * SROWS, SROWS)])


def _sc_segsum(h, src, dst):
    mesh = plsc.VectorSubcoreMesh(core_axis_name="c", subcore_axis_name="s",
                                  num_cores=NC, num_subcores=NS)
    fn = pl.kernel(
        _segsum_body,
        out_type=jax.ShapeDtypeStruct((NC, HR, D), jnp.float32),
        mesh=mesh,
        scratch_types=[
            pltpu.VMEM_SHARED((HR, D), jnp.float32),
            pltpu.VMEM_SHARED((HR, D), jnp.float32),
            pltpu.VMEM((IDXC, CHUNK), jnp.int32),
            pltpu.VMEM((IDXC, CHUNK), jnp.int32),
            [pltpu.VMEM((CHUNK, D), jnp.float32) for _ in range(KBUF)],
            [pltpu.SemaphoreType.DMA for _ in range(KBUF)],
        ],
    )
    return fn(h, src, dst)


def _edge_arrays(edge_index):
    ei = edge_index.astype(jnp.int32)
    goff = (jnp.arange(B, dtype=jnp.int32) % GPC * NP).reshape(B, 1)
    srcl = ei[..., 0] + goff
    dstl = ei[..., 1] + goff
    npad = EPC - GPC * E

    def split(a, padval):
        a2 = a.reshape(NC, GPC * E)
        a2 = jnp.concatenate(
            [a2, jnp.full((NC, npad), padval, jnp.int32)], axis=1)
        return (a2.reshape(NC, NS, EPT).transpose(1, 0, 2)
                  .reshape(NW, NCH, CHUNK))

    return split(srcl, 0), split(dstl, PAD_DST)


def kernel(nodes, edge_index, mask_0, mask_1, W_feat, b_feat, W_upd, b_upd,
           W_glob, b_glob, Wa1, ba1, Wa2, ba2):
    x = jnp.pad(nodes, ((0, 0), (0, NP - N), (0, 0))).reshape(BNP, F)
    src, dst = _edge_arrays(edge_index)

    h = _tc_feat(x, W_feat, b_feat.reshape(1, D))
    for s in range(STEPS):
        parts = _sc_segsum(h, src, dst)
        h = _tc_update(h, parts.reshape(BNP, D), W_upd[s, :D, :],
                       W_upd[s, D:, :], b_upd[s].reshape(1, D))

    h3 = h.reshape(B, NP, D)
    m1 = jnp.pad(mask_1.astype(jnp.float32), ((0, 0), (0, NP - N))
                 ).reshape(B, NP, 1)
    m0 = mask_0.astype(jnp.float32).reshape(B, 1, NUM_ACTIONS)
    out = _tc_heads(h3, m1, m0, Wa1, ba1.reshape(1, 1), Wa2,
                    ba2.reshape(1, NUM_ACTIONS))
    return out.reshape(B, 1 + NUM_ACTIONS)


# speedup vs baseline: 3.2783x; 1.0121x over previous
import jax
import jax.numpy as jnp
from jax import lax
from jax.experimental import pallas as pl
from jax.experimental.pallas import tpu as pltpu
from jax.experimental.pallas import tpu_sc as plsc

B, N, F = 4, 2500, 128
E = 80000
D = 128
NUM_ACTIONS = 8
STEPS = 3

NP = 2560
BNP = B * NP
BE = B * E

NC, NS = 2, 16
NW = NC * NS
GPC = B // NC
HR = GPC * NP
SROWS = HR // NS
CHUNK = 128
NCH = 80
EPT = NCH * CHUNK
EPC = NS * EPT
KBUF = 2
IDXC = 40
PAD_DST = N


def _mish(x):
    sp = jnp.maximum(x, 0.0) + jnp.log1p(jnp.exp(-jnp.abs(x)))
    return x * jnp.tanh(sp)


def _feat_body(x_ref, w_ref, b_ref, o_ref):
    t = jnp.dot(x_ref[0], w_ref[...], preferred_element_type=jnp.float32)
    o_ref[0] = jnp.concatenate(
        [_mish(t + b_ref[...]), jnp.zeros((NP - N, D), jnp.float32)], axis=0)


def _update_body(h_ref, a_ref, w1_ref, w2_ref, b_ref, o_ref):
    t = (jnp.dot(h_ref[...], w1_ref[...], preferred_element_type=jnp.float32)
         + jnp.dot(a_ref[...], w2_ref[...], preferred_element_type=jnp.float32)
         + b_ref[...])
    o_ref[...] = _mish(t)


def _update_heads_body(h_ref, a_ref, w1_ref, w2_ref, b_ref, m1_ref, m0_ref,
                       wa1_ref, ba1_ref, wa2_ref, ba2_ref, o_ref):
    t = (jnp.dot(h_ref[...], w1_ref[...], preferred_element_type=jnp.float32)
         + jnp.dot(a_ref[...], w2_ref[...], preferred_element_type=jnp.float32)
         + b_ref[...])
    hg = _mish(t)
    x1 = jnp.dot(hg, wa1_ref[...], preferred_element_type=jnp.float32)
    x1 = x1 + ba1_ref[0, 0]
    m1 = m1_ref[0]
    x1m = jnp.where(m1 > 0.0, x1, -1e9)
    maxv = jnp.max(x1m)
    iota = lax.broadcasted_iota(jnp.int32, (NP, 1), 0)
    idx = jnp.min(jnp.where(x1m == maxv, iota, jnp.int32(2**30)))
    x2 = jnp.dot(hg, wa2_ref[...], preferred_element_type=jnp.float32)
    x2 = x2 + ba2_ref[...]
    iota2 = lax.broadcasted_iota(jnp.int32, (NP, NUM_ACTIONS), 0)
    sel = jnp.sum(jnp.where(iota2 == idx, x2, 0.0), axis=0, keepdims=True)
    selm = jnp.where(m0_ref[0] > 0.0, sel, -1e9)
    o_ref[0] = jnp.concatenate([maxv.reshape(1, 1), selm], axis=1)


def _tc_feat(x3, w, b2):
    return pl.pallas_call(
        _feat_body,
        grid=(B,),
        in_specs=[
            pl.BlockSpec((1, N, F), lambda i: (i, 0, 0)),
            pl.BlockSpec((F, D), lambda i: (0, 0)),
            pl.BlockSpec((1, D), lambda i: (0, 0)),
        ],
        out_specs=pl.BlockSpec((1, NP, D), lambda i: (i, 0, 0)),
        out_shape=jax.ShapeDtypeStruct((B, NP, D), jnp.float32),
    )(x3, w, b2)


def _tc_update(h, agg, w1, w2, b2):
    return pl.pallas_call(
        _update_body,
        grid=(5,),
        in_specs=[
            pl.BlockSpec((2048, D), lambda i: (i, 0)),
            pl.BlockSpec((2048, D), lambda i: (i, 0)),
            pl.BlockSpec((D, D), lambda i: (0, 0)),
            pl.BlockSpec((D, D), lambda i: (0, 0)),
            pl.BlockSpec((1, D), lambda i: (0, 0)),
        ],
        out_specs=pl.BlockSpec((2048, D), lambda i: (i, 0)),
        out_shape=jax.ShapeDtypeStruct((BNP, D), jnp.float32),
    )(h, agg, w1, w2, b2)


def _tc_update_heads(h, agg, w1, w2, b2, m1, m0, wa1, ba1, wa2, ba2):
    return pl.pallas_call(
        _update_heads_body,
        grid=(B,),
        in_specs=[
            pl.BlockSpec((NP, D), lambda i: (i, 0)),
            pl.BlockSpec((NP, D), lambda i: (i, 0)),
            pl.BlockSpec((D, D), lambda i: (0, 0)),
            pl.BlockSpec((D, D), lambda i: (0, 0)),
            pl.BlockSpec((1, D), lambda i: (0, 0)),
            pl.BlockSpec((1, NP, 1), lambda i: (i, 0, 0)),
            pl.BlockSpec((1, 1, NUM_ACTIONS), lambda i: (i, 0, 0)),
            pl.BlockSpec((D, 1), lambda i: (0, 0)),
            pl.BlockSpec((1, 1), lambda i: (0, 0)),
            pl.BlockSpec((D, NUM_ACTIONS), lambda i: (0, 0)),
            pl.BlockSpec((1, NUM_ACTIONS), lambda i: (0, 0)),
        ],
        out_specs=pl.BlockSpec((1, 1, 1 + NUM_ACTIONS), lambda i: (i, 0, 0)),
        out_shape=jax.ShapeDtypeStruct((B, 1, 1 + NUM_ACTIONS), jnp.float32),
    )(h, agg, w1, w2, b2, m1, m0, wa1, ba1, wa2, ba2)


def _segsum_body(h_hbm, src_hbm, dst_hbm, out_hbm,
                 hloc_sh, agg_sh, srcbuf, dstbuf, rows, sems):
    cid = lax.axis_index("c")
    sid = lax.axis_index("s")
    wid = cid * NS + sid

    hstage = pltpu.async_copy(h_hbm.at[pl.ds(cid * HR + sid * SROWS, SROWS)],
                              hloc_sh.at[pl.ds(sid * SROWS, SROWS)], sems[0])

    zero = jnp.zeros((16,), jnp.float32)

    def zloop(i, _):
        for j in range(D // 16):
            rows[0][i, pl.ds(j * 16, 16)] = zero
        return ()

    lax.fori_loop(0, CHUNK, zloop, ())
    hstage.wait()
    for k in range(SROWS // CHUNK):
        pltpu.sync_copy(rows[0],
                        agg_sh.at[pl.ds(sid * SROWS + k * CHUNK, CHUNK)])
    even = SROWS - (SROWS // CHUNK) * CHUNK
    if even:
        pltpu.sync_copy(
            rows[0].at[pl.ds(0, even)],
            agg_sh.at[pl.ds(sid * SROWS + (SROWS // CHUNK) * CHUNK, even)])
    plsc.subcore_barrier()

    def window(w, _):
        wa = pl.multiple_of(w * IDXC, IDXC)
        pltpu.sync_copy(src_hbm.at[wid, pl.ds(wa, IDXC)], srcbuf)
        pltpu.sync_copy(dst_hbm.at[wid, pl.ds(wa, IDXC)], dstbuf)
        descs = [
            pltpu.async_copy(hloc_sh.at[srcbuf.at[j]], rows[j], sems[j])
            for j in range(KBUF)
        ]
        for j in range(IDXC):
            descs[j].wait()
            pltpu.sync_copy(rows[j % KBUF], agg_sh.at[dstbuf.at[j]], add=True)
            if j + KBUF < IDXC:
                descs.append(
                    pltpu.async_copy(hloc_sh.at[srcbuf.at[j + KBUF]],
                                     rows[(j + KBUF) % KBUF],
                                     sems[(j + KBUF) % KBUF]))
        return ()

    lax.fori_loop(0, NCH // IDXC, window, ())
    plsc.subcore_barrier()

    pltpu.sync_copy(agg_sh.at[pl.ds(sid * SROWS, SROWS)],
                    out_hbm.at[cid].at[pl.ds(sid * SROWS, SROWS)])


def _sc_segsum(h, src, dst):
    mesh = plsc.VectorSubcoreMesh(core_axis_name="c", subcore_axis_name="s",
                                  num_cores=NC, num_subcores=NS)
    fn = pl.kernel(
        _segsum_body,
        out_type=jax.ShapeDtypeStruct((NC, HR, D), jnp.float32),
        mesh=mesh,
        scratch_types=[
            pltpu.VMEM_SHARED((HR, D), jnp.float32),
            pltpu.VMEM_SHARED((HR, D), jnp.float32),
            pltpu.VMEM((IDXC, CHUNK), jnp.int32),
            pltpu.VMEM((IDXC, CHUNK), jnp.int32),
            [pltpu.VMEM((CHUNK, D), jnp.float32) for _ in range(KBUF)],
            [pltpu.SemaphoreType.DMA for _ in range(KBUF)],
        ],
    )
    return fn(h, src, dst)


def _edge_arrays(edge_index):
    ei = edge_index.astype(jnp.int32)
    goff = (jnp.arange(B, dtype=jnp.int32) % GPC * NP).reshape(B, 1)
    srcl = ei[..., 0] + goff
    dstl = ei[..., 1] + goff
    npad = EPC - GPC * E

    def split(a, padval):
        a2 = a.reshape(NC, GPC * E)
        a2 = jnp.concatenate(
            [a2, jnp.full((NC, npad), padval, jnp.int32)], axis=1)
        return a2.reshape(NW, NCH, CHUNK)

    return split(srcl, 0), split(dstl, PAD_DST)


def kernel(nodes, edge_index, mask_0, mask_1, W_feat, b_feat, W_upd, b_upd,
           W_glob, b_glob, Wa1, ba1, Wa2, ba2):
    src, dst = _edge_arrays(edge_index)

    h = _tc_feat(nodes, W_feat, b_feat.reshape(1, D)).reshape(BNP, D)
    for s in range(STEPS - 1):
        parts = _sc_segsum(h, src, dst)
        h = _tc_update(h, parts.reshape(BNP, D), W_upd[s, :D, :],
                       W_upd[s, D:, :], b_upd[s].reshape(1, D))

    parts = _sc_segsum(h, src, dst)
    m1 = jnp.pad(mask_1.astype(jnp.float32), ((0, 0), (0, NP - N))
                 ).reshape(B, NP, 1)
    m0 = mask_0.astype(jnp.float32).reshape(B, 1, NUM_ACTIONS)
    s = STEPS - 1
    out = _tc_update_heads(h, parts.reshape(BNP, D), W_upd[s, :D, :],
                           W_upd[s, D:, :], b_upd[s].reshape(1, D),
                           m1, m0, Wa1, ba1.reshape(1, 1), Wa2,
                           ba2.reshape(1, NUM_ACTIONS))
    return out.reshape(B, 1 + NUM_ACTIONS)
